# Initial kernel scaffold; baseline (speedup 1.0000x reference)
import jax
import jax.numpy as jnp
from jax.experimental import pallas as pl


def kernel(x, edge_index, params):
    raise NotImplementedError("write your pallas kernel here")


# speedup vs baseline: 6.6696x; 6.6696x over previous
import functools

import jax
import jax.numpy as jnp
from jax import lax
from jax.experimental import pallas as pl
from jax.experimental.pallas import tpu as pltpu
from jax.experimental.pallas import tpu_sc as plsc

N = 10000
E = 320000
D = 128

NC = 2
NS = 16
NW = NC * NS
EPT = E // NW
CH = 80
NCHUNK = EPT // CH
NP = 10240
RPT = NP // NS

def _segsum_body(h_hbm, src_hbm, dst_hbm, zeros_hbm, out_hbm,
                 src_v, dst_v, rows_v, aggr_sh, gsem):
    c = lax.axis_index("c")
    s = lax.axis_index("s")
    wid = c * NS + s

    pltpu.sync_copy(zeros_hbm, aggr_sh.at[pl.ds(s * RPT, RPT)])
    pltpu.sync_copy(src_hbm.at[wid], src_v)
    pltpu.sync_copy(dst_hbm.at[wid], dst_v)
    plsc.subcore_barrier()

    def body(j, carry):
        pltpu.async_copy(h_hbm.at[src_v.at[j]], rows_v, gsem).wait()
        pltpu.sync_copy(rows_v, aggr_sh.at[dst_v.at[j]], add=True)
        return carry

    lax.fori_loop(0, NCHUNK, body, 0)

    plsc.subcore_barrier()
    pltpu.sync_copy(aggr_sh.at[pl.ds(s * RPT, RPT)],
                    out_hbm.at[c, pl.ds(s * RPT, RPT)])


@functools.cache
def _segsum_sc():
    mesh = plsc.VectorSubcoreMesh(core_axis_name="c", subcore_axis_name="s",
                                  num_cores=NC, num_subcores=NS)
    return pl.kernel(
        _segsum_body,
        out_type=jax.ShapeDtypeStruct((NC, NP, D), jnp.float32),
        mesh=mesh,
        scratch_types=[
            pltpu.VMEM((NCHUNK, CH), jnp.int32),
            pltpu.VMEM((NCHUNK, CH), jnp.int32),
            pltpu.VMEM((CH, D), jnp.float32),
            pltpu.VMEM_SHARED((NP, D), jnp.float32),
            pltpu.SemaphoreType.DMA,
        ],
    )


def _ln(t, g, b):
    m = jnp.mean(t, axis=-1, keepdims=True)
    v = jnp.mean((t - m) ** 2, axis=-1, keepdims=True)
    return (t - m) * lax.rsqrt(v + 1e-5) * g + b


_V_EPS, _V_BO, _V_BFF = 0, 1, 2
_V_LN1G, _V_LN1B, _V_LN2G, _V_LN2B = 3, 4, 5, 6
_V_ANFG, _V_ANFB, _V_OUTG, _V_OUTB = 7, 8, 9, 10
_V_WA, _V_WB = 11, 12
_NV = 13

_BLK = 1000


def _tc_body(h_ref, pa_ref, pb_ref, wv_ref, wo_ref, wff_ref, vec_ref,
             hout_ref, sacc_ref):
    V = vec_ref[...]

    def row(i):
        return V[i:i + 1, :]

    h = h_ref[...]
    hn = h * (1.0 + row(_V_EPS)) + pa_ref[0] + pb_ref[0]
    o = jnp.dot(jnp.dot(hn, wv_ref[...], preferred_element_type=jnp.float32),
                wo_ref[...], preferred_element_type=jnp.float32) + row(_V_BO)
    h1 = _ln(hn + o, row(_V_LN1G), row(_V_LN1B))
    ff = jnp.maximum(
        jnp.dot(h1, wff_ref[...], preferred_element_type=jnp.float32)
        + row(_V_BFF), 0.0)
    h2 = _ln(h1 + ff, row(_V_LN2G), row(_V_LN2B))
    h3 = jnp.maximum(_ln(h2, row(_V_ANFG), row(_V_ANFB)), 0.0)
    h4 = jnp.maximum(_ln(h3, row(_V_OUTG), row(_V_OUTB)), 0.0)
    hout_ref[...] = h4

    part = (jnp.sum(h * row(_V_WA), axis=0, keepdims=True)
            + jnp.sum(h4 * row(_V_WB), axis=0, keepdims=True))

    @pl.when(pl.program_id(0) == 0)
    def _():
        sacc_ref[...] = jnp.zeros_like(sacc_ref)

    sacc_ref[...] += part


_tc_layer = pl.pallas_call(
    _tc_body,
    grid=(N // _BLK,),
    in_specs=[
        pl.BlockSpec((_BLK, D), lambda i: (i, 0)),
        pl.BlockSpec((1, _BLK, D), lambda i: (0, i, 0)),
        pl.BlockSpec((1, _BLK, D), lambda i: (1, i, 0)),
        pl.BlockSpec((D, D), lambda i: (0, 0)),
        pl.BlockSpec((D, D), lambda i: (0, 0)),
        pl.BlockSpec((D, D), lambda i: (0, 0)),
        pl.BlockSpec((_NV, D), lambda i: (0, 0)),
    ],
    out_specs=[
        pl.BlockSpec((_BLK, D), lambda i: (i, 0)),
        pl.BlockSpec((1, D), lambda i: (0, 0)),
    ],
    out_shape=[
        jax.ShapeDtypeStruct((N, D), jnp.float32),
        jax.ShapeDtypeStruct((1, D), jnp.float32),
    ],
    compiler_params=pltpu.CompilerParams(
        dimension_semantics=("arbitrary",)),
)


def kernel(x, edge_index, params):
    src = edge_index[0].reshape(NW, NCHUNK, CH)
    dst = edge_index[1].reshape(NW, NCHUNK, CH)
    zeros = jnp.zeros((RPT, D), jnp.float32)

    h = x
    score_vecs = []
    for l in range(2):
        p = params['gin%d' % l]
        if l == 0:
            wa = params['pred0']['W'][:, 0]
        else:
            wa = jnp.zeros((D,), jnp.float32)
        wb = params['pred%d' % (l + 1)]['W'][:, 0]
        vecs = jnp.stack([
            jnp.broadcast_to(p['eps'], (D,)), p['bo'], p['bff'],
            p['ln1_g'], p['ln1_b'], p['ln2_g'], p['ln2_b'],
            p['anf_g'], p['anf_b'], p['out_g'], p['out_b'],
            wa, wb,
        ])
        aggr2 = _segsum_sc()(h, src, dst, zeros)
        h, sacc = _tc_layer(h, aggr2, aggr2, p['Wv'], p['Wo'], p['Wff'], vecs)
        score_vecs.append(sacc)

    score = (jnp.sum(score_vecs[0]) + jnp.sum(score_vecs[1])
             + params['pred0']['b'] + params['pred1']['b']
             + params['pred2']['b'])
    return score


# speedup vs baseline: 10.9087x; 1.6356x over previous
import functools

import jax
import jax.numpy as jnp
from jax import lax
from jax.experimental import pallas as pl
from jax.experimental.pallas import tpu as pltpu
from jax.experimental.pallas import tpu_sc as plsc

N = 10000
E = 320000
D = 128

NC = 2
NS = 16
NW = NC * NS
EPT = E // NW
CH = 125
NCHUNK = EPT // CH
NPH = 2
HALF = NCHUNK // NPH
NP = 10240
RPT = NP // NS

def _segsum_body(h_hbm, src_hbm, dst_hbm, zeros_hbm, out_hbm,
                 src_v, dst_v, rows0_v, rows1_v, aggr_sh, gsem0, gsem1):
    c = lax.axis_index("c")
    s = lax.axis_index("s")
    wid = c * NS + s

    pltpu.sync_copy(zeros_hbm, aggr_sh.at[pl.ds(s * RPT, RPT)])
    plsc.subcore_barrier()

    for ph in range(NPH):
        pltpu.sync_copy(src_hbm.at[wid, pl.ds(ph * HALF, HALF)], src_v)
        pltpu.sync_copy(dst_hbm.at[wid, pl.ds(ph * HALF, HALF)], dst_v)
        pltpu.async_copy(h_hbm.at[src_v.at[0]], rows0_v, gsem0)
        pltpu.async_copy(h_hbm.at[src_v.at[1]], rows1_v, gsem1)

        def body(k, carry):
            j = 2 * k
            pltpu.make_async_copy(h_hbm.at[src_v.at[j]], rows0_v, gsem0).wait()
            pltpu.sync_copy(rows0_v, aggr_sh.at[dst_v.at[j]], add=True)

            @pl.when(j + 2 < HALF)
            def _():
                pltpu.async_copy(h_hbm.at[src_v.at[j + 2]], rows0_v, gsem0)

            pltpu.make_async_copy(h_hbm.at[src_v.at[j + 1]], rows1_v, gsem1).wait()
            pltpu.sync_copy(rows1_v, aggr_sh.at[dst_v.at[j + 1]], add=True)

            @pl.when(j + 3 < HALF)
            def _():
                pltpu.async_copy(h_hbm.at[src_v.at[j + 3]], rows1_v, gsem1)

            return carry

        lax.fori_loop(0, HALF // 2, body, 0)

    plsc.subcore_barrier()
    pltpu.sync_copy(aggr_sh.at[pl.ds(s * RPT, RPT)],
                    out_hbm.at[c, pl.ds(s * RPT, RPT)])


@functools.cache
def _segsum_sc():
    mesh = plsc.VectorSubcoreMesh(core_axis_name="c", subcore_axis_name="s",
                                  num_cores=NC, num_subcores=NS)
    return pl.kernel(
        _segsum_body,
        out_type=jax.ShapeDtypeStruct((NC, NP, D), jnp.float32),
        mesh=mesh,
        scratch_types=[
            pltpu.VMEM((HALF, CH), jnp.int32),
            pltpu.VMEM((HALF, CH), jnp.int32),
            pltpu.VMEM((CH, D), jnp.float32),
            pltpu.VMEM((CH, D), jnp.float32),
            pltpu.VMEM_SHARED((NP, D), jnp.float32),
            pltpu.SemaphoreType.DMA,
            pltpu.SemaphoreType.DMA,
        ],
    )


def _ln(t, g, b):
    m = jnp.mean(t, axis=-1, keepdims=True)
    v = jnp.mean((t - m) ** 2, axis=-1, keepdims=True)
    return (t - m) * lax.rsqrt(v + 1e-5) * g + b


_V_EPS, _V_BO, _V_BFF = 0, 1, 2
_V_LN1G, _V_LN1B, _V_LN2G, _V_LN2B = 3, 4, 5, 6
_V_ANFG, _V_ANFB, _V_OUTG, _V_OUTB = 7, 8, 9, 10
_V_WA, _V_WB = 11, 12
_NV = 13

_BLK = 1000


def _tc_body(h_ref, pa_ref, pb_ref, wv_ref, wo_ref, wff_ref, vec_ref,
             hout_ref, sacc_ref):
    V = vec_ref[...]

    def row(i):
        return V[i:i + 1, :]

    h = h_ref[...]
    hn = h * (1.0 + row(_V_EPS)) + pa_ref[0] + pb_ref[0]
    o = jnp.dot(jnp.dot(hn, wv_ref[...], preferred_element_type=jnp.float32),
                wo_ref[...], preferred_element_type=jnp.float32) + row(_V_BO)
    h1 = _ln(hn + o, row(_V_LN1G), row(_V_LN1B))
    ff = jnp.maximum(
        jnp.dot(h1, wff_ref[...], preferred_element_type=jnp.float32)
        + row(_V_BFF), 0.0)
    h2 = _ln(h1 + ff, row(_V_LN2G), row(_V_LN2B))
    h3 = jnp.maximum(_ln(h2, row(_V_ANFG), row(_V_ANFB)), 0.0)
    h4 = jnp.maximum(_ln(h3, row(_V_OUTG), row(_V_OUTB)), 0.0)
    hout_ref[...] = h4

    part = (jnp.sum(h * row(_V_WA), axis=0, keepdims=True)
            + jnp.sum(h4 * row(_V_WB), axis=0, keepdims=True))

    @pl.when(pl.program_id(0) == 0)
    def _():
        sacc_ref[...] = jnp.zeros_like(sacc_ref)

    sacc_ref[...] += part


_tc_layer = pl.pallas_call(
    _tc_body,
    grid=(N // _BLK,),
    in_specs=[
        pl.BlockSpec((_BLK, D), lambda i: (i, 0)),
        pl.BlockSpec((1, _BLK, D), lambda i: (0, i, 0)),
        pl.BlockSpec((1, _BLK, D), lambda i: (1, i, 0)),
        pl.BlockSpec((D, D), lambda i: (0, 0)),
        pl.BlockSpec((D, D), lambda i: (0, 0)),
        pl.BlockSpec((D, D), lambda i: (0, 0)),
        pl.BlockSpec((_NV, D), lambda i: (0, 0)),
    ],
    out_specs=[
        pl.BlockSpec((_BLK, D), lambda i: (i, 0)),
        pl.BlockSpec((1, D), lambda i: (0, 0)),
    ],
    out_shape=[
        jax.ShapeDtypeStruct((N, D), jnp.float32),
        jax.ShapeDtypeStruct((1, D), jnp.float32),
    ],
    compiler_params=pltpu.CompilerParams(
        dimension_semantics=("arbitrary",)),
)


def kernel(x, edge_index, params):
    src = edge_index[0].reshape(NW, NCHUNK, CH)
    dst = edge_index[1].reshape(NW, NCHUNK, CH)
    zeros = jnp.zeros((RPT, D), jnp.float32)

    h = x
    score_vecs = []
    for l in range(2):
        p = params['gin%d' % l]
        if l == 0:
            wa = params['pred0']['W'][:, 0]
        else:
            wa = jnp.zeros((D,), jnp.float32)
        wb = params['pred%d' % (l + 1)]['W'][:, 0]
        vecs = jnp.stack([
            jnp.broadcast_to(p['eps'], (D,)), p['bo'], p['bff'],
            p['ln1_g'], p['ln1_b'], p['ln2_g'], p['ln2_b'],
            p['anf_g'], p['anf_b'], p['out_g'], p['out_b'],
            wa, wb,
        ])
        aggr2 = _segsum_sc()(h, src, dst, zeros)
        h, sacc = _tc_layer(h, aggr2, aggr2, p['Wv'], p['Wo'], p['Wff'], vecs)
        score_vecs.append(sacc)

    score = (jnp.sum(score_vecs[0]) + jnp.sum(score_vecs[1])
             + params['pred0']['b'] + params['pred1']['b']
             + params['pred2']['b'])
    return score


# speedup vs baseline: 11.4781x; 1.0522x over previous
import functools

import jax
import jax.numpy as jnp
from jax import lax
from jax.experimental import pallas as pl
from jax.experimental.pallas import tpu as pltpu
from jax.experimental.pallas import tpu_sc as plsc

N = 10000
E = 320000
D = 128

NC = 2
NS = 16
NW = NC * NS
EPT = E // NW
CH = 125
NCHUNK = EPT // CH
NPH = 2
HALF = NCHUNK // NPH
NP = 10240
RPT = NP // NS

def _segsum_body(h_hbm, edge_hbm, zeros_hbm, out_hbm,
                 src_v, dst_v, rows0_v, rows1_v, aggr_sh, gsem0, gsem1):
    c = lax.axis_index("c")
    s = lax.axis_index("s")
    wid = c * NS + s

    pltpu.sync_copy(zeros_hbm, aggr_sh.at[pl.ds(s * RPT, RPT)])
    plsc.subcore_barrier()

    for ph in range(NPH):
        pltpu.sync_copy(edge_hbm.at[0, wid, pl.ds(ph * HALF, HALF)], src_v)
        pltpu.sync_copy(edge_hbm.at[1, wid, pl.ds(ph * HALF, HALF)], dst_v)
        pltpu.async_copy(h_hbm.at[src_v.at[0]], rows0_v, gsem0)
        pltpu.async_copy(h_hbm.at[src_v.at[1]], rows1_v, gsem1)

        def body(k, carry):
            j = 2 * k
            pltpu.make_async_copy(h_hbm.at[src_v.at[j]], rows0_v, gsem0).wait()
            pltpu.sync_copy(rows0_v, aggr_sh.at[dst_v.at[j]], add=True)

            @pl.when(j + 2 < HALF)
            def _():
                pltpu.async_copy(h_hbm.at[src_v.at[j + 2]], rows0_v, gsem0)

            pltpu.make_async_copy(h_hbm.at[src_v.at[j + 1]], rows1_v, gsem1).wait()
            pltpu.sync_copy(rows1_v, aggr_sh.at[dst_v.at[j + 1]], add=True)

            @pl.when(j + 3 < HALF)
            def _():
                pltpu.async_copy(h_hbm.at[src_v.at[j + 3]], rows1_v, gsem1)

            return carry

        lax.fori_loop(0, HALF // 2, body, 0)

    plsc.subcore_barrier()
    pltpu.sync_copy(aggr_sh.at[pl.ds(s * RPT, RPT)],
                    out_hbm.at[c, pl.ds(s * RPT, RPT)])


@functools.cache
def _segsum_sc():
    mesh = plsc.VectorSubcoreMesh(core_axis_name="c", subcore_axis_name="s",
                                  num_cores=NC, num_subcores=NS)
    return pl.kernel(
        _segsum_body,
        out_type=jax.ShapeDtypeStruct((NC, NP, D), jnp.float32),
        mesh=mesh,
        scratch_types=[
            pltpu.VMEM((HALF, CH), jnp.int32),
            pltpu.VMEM((HALF, CH), jnp.int32),
            pltpu.VMEM((CH, D), jnp.float32),
            pltpu.VMEM((CH, D), jnp.float32),
            pltpu.VMEM_SHARED((NP, D), jnp.float32),
            pltpu.SemaphoreType.DMA,
            pltpu.SemaphoreType.DMA,
        ],
    )


def _ln(t, g, b):
    m = jnp.mean(t, axis=-1, keepdims=True)
    v = jnp.mean((t - m) ** 2, axis=-1, keepdims=True)
    return (t - m) * lax.rsqrt(v + 1e-5) * g + b


_V_EPS, _V_BO, _V_BFF = 0, 1, 2
_V_LN1G, _V_LN1B, _V_LN2G, _V_LN2B = 3, 4, 5, 6
_V_ANFG, _V_ANFB, _V_OUTG, _V_OUTB = 7, 8, 9, 10
_NV = 11

_BLK = 2000


def _tc_body(h_ref, pa_ref, pb_ref, wv_ref, wo_ref, wff_ref, vec_ref,
             wa_ref, wb_ref, hout_ref, sacc_ref):
    V = vec_ref[...]

    def row(i):
        return V[i:i + 1, :]

    h = h_ref[...]
    hn = h * (1.0 + row(_V_EPS)) + pa_ref[0] + pb_ref[0]
    wvo = jnp.dot(wv_ref[...], wo_ref[...], preferred_element_type=jnp.float32)
    o = jnp.dot(hn, wvo, preferred_element_type=jnp.float32) + row(_V_BO)
    h1 = _ln(hn + o, row(_V_LN1G), row(_V_LN1B))
    ff = jnp.maximum(
        jnp.dot(h1, wff_ref[...], preferred_element_type=jnp.float32)
        + row(_V_BFF), 0.0)
    h2 = _ln(h1 + ff, row(_V_LN2G), row(_V_LN2B))
    h3 = jnp.maximum(_ln(h2, row(_V_ANFG), row(_V_ANFB)), 0.0)
    h4 = jnp.maximum(_ln(h3, row(_V_OUTG), row(_V_OUTB)), 0.0)
    hout_ref[...] = h4

    part = (jnp.sum(jnp.dot(h, wa_ref[...], preferred_element_type=jnp.float32))
            + jnp.sum(jnp.dot(h4, wb_ref[...], preferred_element_type=jnp.float32)))

    @pl.when(pl.program_id(0) == 0)
    def _():
        sacc_ref[...] = jnp.zeros_like(sacc_ref)

    sacc_ref[...] += jnp.reshape(part, (1, 1))


_tc_layer = pl.pallas_call(
    _tc_body,
    grid=(N // _BLK,),
    in_specs=[
        pl.BlockSpec((_BLK, D), lambda i: (i, 0)),
        pl.BlockSpec((1, _BLK, D), lambda i: (0, i, 0)),
        pl.BlockSpec((1, _BLK, D), lambda i: (1, i, 0)),
        pl.BlockSpec((D, D), lambda i: (0, 0)),
        pl.BlockSpec((D, D), lambda i: (0, 0)),
        pl.BlockSpec((D, D), lambda i: (0, 0)),
        pl.BlockSpec((_NV, D), lambda i: (0, 0)),
        pl.BlockSpec((D, 1), lambda i: (0, 0)),
        pl.BlockSpec((D, 1), lambda i: (0, 0)),
    ],
    out_specs=[
        pl.BlockSpec((_BLK, D), lambda i: (i, 0)),
        pl.BlockSpec((1, 1), lambda i: (0, 0)),
    ],
    out_shape=[
        jax.ShapeDtypeStruct((N, D), jnp.float32),
        jax.ShapeDtypeStruct((1, 1), jnp.float32),
    ],
    compiler_params=pltpu.CompilerParams(
        dimension_semantics=("arbitrary",)),
)


def kernel(x, edge_index, params):
    edge_r = edge_index.reshape(2, NW, NCHUNK, CH)
    zeros = jnp.zeros((RPT, D), jnp.float32)
    wzero = jnp.zeros((D, 1), jnp.float32)

    h = x
    saccs = []
    for l in range(2):
        p = params['gin%d' % l]
        wa = params['pred0']['W'] if l == 0 else wzero
        wb = params['pred%d' % (l + 1)]['W']
        vecs = jnp.stack([
            jnp.broadcast_to(p['eps'], (D,)), p['bo'], p['bff'],
            p['ln1_g'], p['ln1_b'], p['ln2_g'], p['ln2_b'],
            p['anf_g'], p['anf_b'], p['out_g'], p['out_b'],
        ])
        aggr2 = _segsum_sc()(h, edge_r, zeros)
        h, sacc = _tc_layer(h, aggr2, aggr2, p['Wv'], p['Wo'], p['Wff'],
                            vecs, wa, wb)
        saccs.append(sacc)

    score = (saccs[0].reshape((1,)) + saccs[1].reshape((1,))
             + params['pred0']['b'] + params['pred1']['b']
             + params['pred2']['b'])
    return score
